# baseline (device time: 158491 ns/iter reference)
import jax
import jax.numpy as jnp
from jax import lax
from jax.experimental import pallas as pl
from jax.experimental.pallas import tpu as pltpu

N = 8
B, S, D = 2, 512, 2048
H, Dh, Dr = 16, 128, 32
HB = 256
RB = 64
F32 = jnp.float32
BF16 = jnp.bfloat16
SCALE = (Dh + Dr) ** -0.5


def _body(x_ref, wdkv_ref, wuk_ref, wuv_ref, wq_ref, wqr_ref, wkr_ref,
          wo_ref, out_ref,
          c_ref, xb_ref, ksend_ref, vsend_ref, rbk_ref, rbv_ref,
          kfin_ref, vfin_ref,
          osched_ref, q_ref, qr_ref, kr_ref,
          kf_send, kf_recv, vb_send, vb_recv,
          agf_send, agf_recv, agb_send, agb_recv):
    my = lax.axis_index("i")
    left = (my + N - 1) % N
    right = (my + 1) % N
    own = right

    def k_partial(t):
        for b in range(B):
            kfin_ref[b] = jnp.dot(
                c_ref[b], wuk_ref[:, t * HB:(t + 1) * HB],
                preferred_element_type=F32)

    def v_partial(t):
        for b in range(B):
            vfin_ref[b] = jnp.dot(
                c_ref[b], wuv_ref[:, t * HB:(t + 1) * HB],
                preferred_element_type=F32)

    for b in range(B):
        c_ref[b] = jnp.dot(x_ref[b], wdkv_ref[...], preferred_element_type=F32)
    k_partial(0)
    v_partial(0)
    for b in range(B):
        ksend_ref[0, b] = kfin_ref[b].astype(BF16)
        vsend_ref[0, b] = vfin_ref[b].astype(BF16)

    barrier_sem = pltpu.get_barrier_semaphore()
    for nbr in (left, right):
        pl.semaphore_signal(barrier_sem, inc=1, device_id=(nbr,),
                            device_id_type=pl.DeviceIdType.MESH)
    pl.semaphore_wait(barrier_sem, 2)

    for s in range(N - 1):
        rk = pltpu.make_async_remote_copy(
            src_ref=ksend_ref.at[s % 2], dst_ref=rbk_ref.at[s],
            send_sem=kf_send.at[s], recv_sem=kf_recv.at[s],
            device_id=(right,), device_id_type=pl.DeviceIdType.MESH)
        rv = pltpu.make_async_remote_copy(
            src_ref=vsend_ref.at[s % 2], dst_ref=rbv_ref.at[s],
            send_sem=vb_send.at[s], recv_sem=vb_recv.at[s],
            device_id=(left,), device_id_type=pl.DeviceIdType.MESH)
        rk.start()
        rv.start()
        if s == 0:
            for b in range(B):
                xb_ref[b] = x_ref[b].astype(BF16)
        if s == 1:
            for b in range(B):
                q_ref[b] = jnp.dot(xb_ref[b], wq_ref[...],
                                   preferred_element_type=F32).astype(BF16)
        if s == 2:
            for b in range(B):
                qr_ref[b] = jnp.dot(xb_ref[b], wqr_ref[...],
                                    preferred_element_type=F32).astype(BF16)
                kr_ref[b] = jnp.dot(xb_ref[b], wkr_ref[...],
                                    preferred_element_type=F32).astype(BF16)
        k_partial(s + 1)
        v_partial(s + 1)
        rk.wait()
        rv.wait()
        last = s == N - 2
        for b in range(B):
            kacc = kfin_ref[b] + rbk_ref[s, b].astype(F32)
            vacc = vfin_ref[b] + rbv_ref[s, b].astype(F32)
            if last:
                kfin_ref[b] = kacc
                vfin_ref[b] = vacc
            else:
                ksend_ref[(s + 1) % 2, b] = kacc.astype(BF16)
                vsend_ref[(s + 1) % 2, b] = vacc.astype(BF16)

    def attn_head(h, col0):
        for b in range(B):
            qh = q_ref[b, :, h * Dh:(h + 1) * Dh]
            kh = kfin_ref[b, :, h * Dh:(h + 1) * Dh].astype(BF16)
            vh = vfin_ref[b, :, h * Dh:(h + 1) * Dh].astype(BF16)
            qrh = qr_ref[b, :, h * Dr:(h + 1) * Dr]
            krb = kr_ref[b]
            sc = lax.dot_general(qh, kh, (((1,), (1,)), ((), ())),
                                 preferred_element_type=F32)
            sc = sc + lax.dot_general(qrh, krb, (((1,), (1,)), ((), ())),
                                      preferred_element_type=F32)
            p = jnp.exp(sc * SCALE)
            p = (p / jnp.sum(p, axis=-1, keepdims=True)).astype(BF16)
            osched_ref[b, :, col0:col0 + Dh] = jnp.dot(
                p, vh, preferred_element_type=F32).astype(BF16)

    def fwd_rdma(hop):
        return pltpu.make_async_remote_copy(
            src_ref=osched_ref.at[:, :, hop * Dh:(hop + 1) * Dh],
            dst_ref=osched_ref.at[:, :, (hop + 1) * Dh:(hop + 2) * Dh],
            send_sem=agf_send.at[hop], recv_sem=agf_recv.at[hop],
            device_id=(right,), device_id_type=pl.DeviceIdType.MESH)

    def bwd_rdma(hop):
        c0 = N * Dh
        return pltpu.make_async_remote_copy(
            src_ref=osched_ref.at[:, :, c0 + hop * Dh:c0 + (hop + 1) * Dh],
            dst_ref=osched_ref.at[:, :, c0 + (hop + 1) * Dh:
                                  c0 + (hop + 2) * Dh],
            send_sem=agb_send.at[hop], recv_sem=agb_recv.at[hop],
            device_id=(left,), device_id_type=pl.DeviceIdType.MESH)

    attn_head(0, 0)
    fwd_rdma(0).start()
    attn_head(1, N * Dh)

    def proj_pass(k):
        f0, b0 = 2 * k * Dh, N * Dh + 2 * k * Dh
        for b in range(B):
            o = jnp.dot(osched_ref[b, :, f0:f0 + 2 * Dh],
                        wo_ref[f0:f0 + 2 * Dh, :],
                        preferred_element_type=F32)
            o = o + jnp.dot(osched_ref[b, :, b0:b0 + 2 * Dh],
                            wo_ref[b0:b0 + 2 * Dh, :],
                            preferred_element_type=F32)
            if k == 0:
                out_ref[b] = o
            else:
                out_ref[b] = out_ref[b] + o

    for hop in range(N - 1):
        rf = fwd_rdma(hop)
        rb = bwd_rdma(hop)
        if hop > 0:
            rf.start()
        rb.start()
        if hop % 2 == 1:
            proj_pass(hop // 2)
        rf.wait()
        rb.wait()
    proj_pass(3)


def kernel(x, Wdkv, Wuk, Wuv, Wq, Wqr, Wkr, Wo):
    my = lax.axis_index("i")
    own = (my + 1) % N
    Wq_own = lax.dynamic_slice(Wq, (0, own * HB), (D, HB)).astype(BF16)
    Wqr_own = lax.dynamic_slice(Wqr, (0, own * RB), (D, RB)).astype(BF16)
    Wkr_bf = Wkr.astype(BF16)
    perm_k = (my - jnp.arange(N)) % N
    perm_v = (my + 2 + jnp.arange(N)) % N
    Wuk_p = jnp.take(Wuk.reshape(128, N, HB), perm_k, axis=1).reshape(128, D)
    Wuv_p = jnp.take(Wuv.reshape(128, N, HB), perm_v, axis=1).reshape(128, D)
    t_arr = jnp.arange(N)
    r_arr = jnp.arange(Dh)
    fwd_rows = (((my + 1 - t_arr) % N) * HB)[:, None] + r_arr[None, :]
    bwd_rows = (((my + 1 + t_arr) % N) * HB + Dh)[:, None] + r_arr[None, :]
    row_idx = jnp.concatenate(
        [fwd_rows.reshape(-1), bwd_rows.reshape(-1)])
    Wo_p = jnp.take(Wo.astype(BF16), row_idx, axis=0)

    return pl.pallas_call(
        _body,
        out_shape=jax.ShapeDtypeStruct((B, S, D), F32),
        in_specs=[pl.BlockSpec(memory_space=pltpu.VMEM)] * 8,
        out_specs=pl.BlockSpec(memory_space=pltpu.VMEM),
        scratch_shapes=[
            pltpu.VMEM((B, S, 128), F32),
            pltpu.VMEM((B, S, D), BF16),
            pltpu.VMEM((2, B, S, HB), BF16),
            pltpu.VMEM((2, B, S, HB), BF16),
            pltpu.VMEM((N - 1, B, S, HB), BF16),
            pltpu.VMEM((N - 1, B, S, HB), BF16),
            pltpu.VMEM((B, S, HB), F32),
            pltpu.VMEM((B, S, HB), F32),
            pltpu.VMEM((B, S, 2 * N * Dh), BF16),
            pltpu.VMEM((B, S, HB), BF16),
            pltpu.VMEM((B, S, RB), BF16),
            pltpu.VMEM((B, S, Dr), BF16),
            pltpu.SemaphoreType.DMA((N - 1,)),
            pltpu.SemaphoreType.DMA((N - 1,)),
            pltpu.SemaphoreType.DMA((N - 1,)),
            pltpu.SemaphoreType.DMA((N - 1,)),
            pltpu.SemaphoreType.DMA((N - 1,)),
            pltpu.SemaphoreType.DMA((N - 1,)),
            pltpu.SemaphoreType.DMA((N - 1,)),
            pltpu.SemaphoreType.DMA((N - 1,)),
        ],
        compiler_params=pltpu.CompilerParams(
            collective_id=0, vmem_limit_bytes=100 * 1024 * 1024),
    )(x, Wdkv, Wuk_p, Wuv_p, Wq_own, Wqr_own, Wkr_bf, Wo_p)


# device time: 137354 ns/iter; 1.1539x vs baseline; 1.1539x over previous
import jax
import jax.numpy as jnp
from jax import lax
from jax.experimental import pallas as pl
from jax.experimental.pallas import tpu as pltpu

N = 8
B, S, D = 2, 512, 2048
H, Dh, Dr = 16, 128, 32
HB = 256
RB = 64
F32 = jnp.float32
BF16 = jnp.bfloat16
SCALE = (Dh + Dr) ** -0.5


def _body(x_ref, wdkv_ref, wuk_ref, wuv_ref, wq_ref, wqr_ref, wkr_ref,
          wo_ref, out_ref,
          c_ref, xb_ref, ksend_ref, vsend_ref, rbk_ref, rbv_ref,
          kfin_ref, vfin_ref,
          obf_ref, obb_ref, q_ref, qr_ref, kr_ref, wof_buf, wob_buf,
          kf_send, kf_recv, vb_send, vb_recv,
          agf_send, agf_recv, agb_send, agb_recv, wof_sems, wob_sems):
    my = lax.axis_index("i")
    left = (my + N - 1) % N
    right = (my + 1) % N
    own = right

    def k_partial(t):
        for b in range(B):
            kfin_ref[b] = jnp.dot(
                c_ref[b], wuk_ref[:, t * HB:(t + 1) * HB],
                preferred_element_type=F32)

    def v_partial(t):
        for b in range(B):
            vfin_ref[b] = jnp.dot(
                c_ref[b], wuv_ref[:, t * HB:(t + 1) * HB],
                preferred_element_type=F32)

    def wof_fetch(t, slot):
        blk = (my + 1 - t) % N
        return pltpu.make_async_copy(
            wo_ref.at[pl.ds(blk * HB, Dh), :], wof_buf.at[slot],
            wof_sems.at[slot])

    def wob_fetch(t, slot):
        blk = (my + 1 + t) % N
        return pltpu.make_async_copy(
            wo_ref.at[pl.ds(blk * HB + Dh, Dh), :], wob_buf.at[slot],
            wob_sems.at[slot])

    wof_fetch(0, 0).start()
    wob_fetch(0, 0).start()
    wof_fetch(1, 1).start()
    wob_fetch(1, 1).start()

    for b in range(B):
        c_ref[b] = jnp.dot(x_ref[b], wdkv_ref[...], preferred_element_type=F32)
    k_partial(0)
    v_partial(0)
    for b in range(B):
        ksend_ref[0, b] = kfin_ref[b].astype(BF16)
        vsend_ref[0, b] = vfin_ref[b].astype(BF16)

    barrier_sem = pltpu.get_barrier_semaphore()
    for nbr in (left, right):
        pl.semaphore_signal(barrier_sem, inc=1, device_id=(nbr,),
                            device_id_type=pl.DeviceIdType.MESH)
    pl.semaphore_wait(barrier_sem, 2)

    for s in range(N - 1):
        rk = pltpu.make_async_remote_copy(
            src_ref=ksend_ref.at[s % 2], dst_ref=rbk_ref.at[s],
            send_sem=kf_send.at[s], recv_sem=kf_recv.at[s],
            device_id=(right,), device_id_type=pl.DeviceIdType.MESH)
        rv = pltpu.make_async_remote_copy(
            src_ref=vsend_ref.at[s % 2], dst_ref=rbv_ref.at[s],
            send_sem=vb_send.at[s], recv_sem=vb_recv.at[s],
            device_id=(left,), device_id_type=pl.DeviceIdType.MESH)
        rk.start()
        rv.start()
        if s == 0:
            for b in range(B):
                xb_ref[b] = x_ref[b].astype(BF16)
        if s == 1:
            for b in range(B):
                q_ref[b] = jnp.dot(xb_ref[b], wq_ref[...],
                                   preferred_element_type=F32).astype(BF16)
        if s == 2:
            for b in range(B):
                qr_ref[b] = jnp.dot(xb_ref[b], wqr_ref[...],
                                    preferred_element_type=F32).astype(BF16)
                kr_ref[b] = jnp.dot(xb_ref[b], wkr_ref[...],
                                    preferred_element_type=F32).astype(BF16)
        k_partial(s + 1)
        v_partial(s + 1)
        rk.wait()
        rv.wait()
        last = s == N - 2
        for b in range(B):
            kacc = kfin_ref[b] + rbk_ref[s, b].astype(F32)
            vacc = vfin_ref[b] + rbv_ref[s, b].astype(F32)
            if last:
                kfin_ref[b] = kacc
                vfin_ref[b] = vacc
            else:
                ksend_ref[(s + 1) % 2, b] = kacc.astype(BF16)
                vsend_ref[(s + 1) % 2, b] = vacc.astype(BF16)

    def attn_head(h, dst_ref):
        for b in range(B):
            qh = q_ref[b, :, h * Dh:(h + 1) * Dh]
            kh = kfin_ref[b, :, h * Dh:(h + 1) * Dh].astype(BF16)
            vh = vfin_ref[b, :, h * Dh:(h + 1) * Dh].astype(BF16)
            qrh = qr_ref[b, :, h * Dr:(h + 1) * Dr]
            krb = kr_ref[b]
            sc = lax.dot_general(qh, kh, (((1,), (1,)), ((), ())),
                                 preferred_element_type=F32)
            sc = sc + lax.dot_general(qrh, krb, (((1,), (1,)), ((), ())),
                                      preferred_element_type=F32)
            p = jnp.exp(sc * SCALE)
            p = (p / jnp.sum(p, axis=-1, keepdims=True)).astype(BF16)
            dst_ref[own, b] = jnp.dot(
                p, vh, preferred_element_type=F32).astype(BF16)

    def fwd_rdma(hop):
        sblk = (my + 1 - hop) % N
        return pltpu.make_async_remote_copy(
            src_ref=obf_ref.at[sblk], dst_ref=obf_ref.at[sblk],
            send_sem=agf_send.at[hop], recv_sem=agf_recv.at[hop],
            device_id=(right,), device_id_type=pl.DeviceIdType.MESH)

    def bwd_rdma(hop):
        sblk = (my + 1 + hop) % N
        return pltpu.make_async_remote_copy(
            src_ref=obb_ref.at[sblk], dst_ref=obb_ref.at[sblk],
            send_sem=agb_send.at[hop], recv_sem=agb_recv.at[hop],
            device_id=(left,), device_id_type=pl.DeviceIdType.MESH)

    attn_head(0, obf_ref)
    fwd_rdma(0).start()
    attn_head(1, obb_ref)

    def proj_f(t, slot):
        blk = (my + 1 - t) % N
        wof_fetch(t, slot).wait()
        for b in range(B):
            o = jnp.dot(obf_ref[blk, b], wof_buf[slot],
                        preferred_element_type=F32)
            if t == 0:
                out_ref[b] = o
            else:
                out_ref[b] = out_ref[b] + o

    def proj_b(t, slot):
        blk = (my + 1 + t) % N
        wob_fetch(t, slot).wait()
        for b in range(B):
            out_ref[b] = out_ref[b] + jnp.dot(
                obb_ref[blk, b], wob_buf[slot], preferred_element_type=F32)

    for hop in range(N - 1):
        rf = fwd_rdma(hop)
        rb = bwd_rdma(hop)
        if hop > 0:
            rf.start()
        rb.start()
        proj_f(hop, hop % 2)
        proj_b(hop, hop % 2)
        if hop + 2 < N:
            wof_fetch(hop + 2, hop % 2).start()
            wob_fetch(hop + 2, hop % 2).start()
        rf.wait()
        rb.wait()
    proj_f(N - 1, (N - 1) % 2)
    proj_b(N - 1, (N - 1) % 2)


def kernel(x, Wdkv, Wuk, Wuv, Wq, Wqr, Wkr, Wo):
    my = lax.axis_index("i")
    own = (my + 1) % N
    Wq_own = lax.dynamic_slice(Wq, (0, own * HB), (D, HB)).astype(BF16)
    Wqr_own = lax.dynamic_slice(Wqr, (0, own * RB), (D, RB)).astype(BF16)
    Wkr_bf = Wkr.astype(BF16)
    perm_k = (my - jnp.arange(N)) % N
    perm_v = (my + 2 + jnp.arange(N)) % N
    Wuk_p = jnp.take(Wuk.reshape(128, N, HB), perm_k, axis=1).reshape(128, D)
    Wuv_p = jnp.take(Wuv.reshape(128, N, HB), perm_v, axis=1).reshape(128, D)
    Wo_bf = Wo.astype(BF16)

    return pl.pallas_call(
        _body,
        out_shape=jax.ShapeDtypeStruct((B, S, D), F32),
        in_specs=[pl.BlockSpec(memory_space=pltpu.VMEM)] * 7
        + [pl.BlockSpec(memory_space=pl.ANY)],
        out_specs=pl.BlockSpec(memory_space=pltpu.VMEM),
        scratch_shapes=[
            pltpu.VMEM((B, S, 128), F32),
            pltpu.VMEM((B, S, D), BF16),
            pltpu.VMEM((2, B, S, HB), BF16),
            pltpu.VMEM((2, B, S, HB), BF16),
            pltpu.VMEM((N - 1, B, S, HB), BF16),
            pltpu.VMEM((N - 1, B, S, HB), BF16),
            pltpu.VMEM((B, S, HB), F32),
            pltpu.VMEM((B, S, HB), F32),
            pltpu.VMEM((N, B, S, Dh), BF16),
            pltpu.VMEM((N, B, S, Dh), BF16),
            pltpu.VMEM((B, S, HB), BF16),
            pltpu.VMEM((B, S, RB), BF16),
            pltpu.VMEM((B, S, Dr), BF16),
            pltpu.VMEM((2, Dh, D), BF16),
            pltpu.VMEM((2, Dh, D), BF16),
            pltpu.SemaphoreType.DMA((N - 1,)),
            pltpu.SemaphoreType.DMA((N - 1,)),
            pltpu.SemaphoreType.DMA((N - 1,)),
            pltpu.SemaphoreType.DMA((N - 1,)),
            pltpu.SemaphoreType.DMA((N - 1,)),
            pltpu.SemaphoreType.DMA((N - 1,)),
            pltpu.SemaphoreType.DMA((N - 1,)),
            pltpu.SemaphoreType.DMA((N - 1,)),
            pltpu.SemaphoreType.DMA((2,)),
            pltpu.SemaphoreType.DMA((2,)),
        ],
        compiler_params=pltpu.CompilerParams(
            collective_id=0, vmem_limit_bytes=100 * 1024 * 1024),
    )(x, Wdkv, Wuk_p, Wuv_p, Wq_own, Wqr_own, Wkr_bf, Wo_bf)
